# baseline (speedup 1.0000x reference)
import functools

import jax
import jax.numpy as jnp
from jax import lax
from jax.experimental import pallas as pl
from jax.experimental.pallas import tpu as pltpu
from jax.experimental.pallas import tpu_sc as plsc

N_E = 8192
E_DIM = 256
BETA = 0.25
B, C, H = 8, 256, 1024
TOK = B * H

KB = 512
WIN = 4096
KSTEPS = WIN // KB
MM_PREC = lax.Precision.DEFAULT


def _argmin_body(offs_ref, width_ref, z_ref, emb_ref, w_ref, b_ref, idx_ref,
                 qc_ref, zn_ref, rmin_ref, ridx_ref):
    b = pl.program_id(0)
    k = pl.program_id(1)

    @pl.when((b == 0) & (k == 0))
    def _project():
        qc_ref[...] = lax.dot_general(
            emb_ref[...], w_ref[...], (((1,), (1,)), ((), ())),
            precision=MM_PREC,
            preferred_element_type=jnp.float32) + b_ref[...]

    @pl.when(k == 0)
    def _init():
        z_b = z_ref[0]
        zn_ref[...] = jnp.sum(z_b * z_b, axis=0, keepdims=True)
        rmin_ref[...] = jnp.full((1, H), jnp.inf, jnp.float32)
        ridx_ref[...] = jnp.zeros((1, H), jnp.float32)

    @pl.when(k * KB < width_ref[b])
    def _compute():
        start = (offs_ref[b] + k) * KB
        qc = qc_ref[pl.ds(start, KB), :]
        cbn = jnp.sum(qc * qc, axis=1, keepdims=True)
        mm = lax.dot_general(qc, z_ref[0], (((1,), (0,)), ((), ())),
                             precision=MM_PREC,
                             preferred_element_type=jnp.float32)
        d = (zn_ref[...] + cbn) - 2.0 * mm

        rows = (jnp.float32(k * KB)
                + lax.broadcasted_iota(jnp.int32, (KB, 1), 0).astype(jnp.float32))
        cmin = jnp.min(d, axis=0, keepdims=True)
        cidx = jnp.min(jnp.where(d == cmin, rows, jnp.float32(2**30)),
                       axis=0, keepdims=True)

        take = cmin < rmin_ref[...]
        rmin_ref[...] = jnp.where(take, cmin, rmin_ref[...])
        ridx_ref[...] = jnp.where(take, cidx, ridx_ref[...])

    @pl.when(k == KSTEPS - 1)
    def _emit():
        idx_ref[0] = ridx_ref[...].astype(jnp.int32) + offs_ref[b] * KB


def _search(z, embedding, proj_W, proj_b, offs, width):
    grid_spec = pltpu.PrefetchScalarGridSpec(
        num_scalar_prefetch=2,
        grid=(B, KSTEPS),
        in_specs=[
            pl.BlockSpec((1, C, H), lambda b, k, offs, width: (b, 0, 0)),
            pl.BlockSpec((N_E, E_DIM), lambda b, k, offs, width: (0, 0)),
            pl.BlockSpec((E_DIM, E_DIM), lambda b, k, offs, width: (0, 0)),
            pl.BlockSpec((1, E_DIM), lambda b, k, offs, width: (0, 0)),
        ],
        out_specs=pl.BlockSpec((1, 1, H), lambda b, k, offs, width: (b, 0, 0)),
        scratch_shapes=[
            pltpu.VMEM((N_E, E_DIM), jnp.float32),
            pltpu.VMEM((1, H), jnp.float32),
            pltpu.VMEM((1, H), jnp.float32),
            pltpu.VMEM((1, H), jnp.float32),
        ],
    )
    return pl.pallas_call(
        _argmin_body,
        grid_spec=grid_spec,
        out_shape=jax.ShapeDtypeStruct((B, 1, H), jnp.int32),
    )(offs, width, z, embedding, proj_W, proj_b.reshape(1, E_DIM))


_CHUNK = 128


def _sc_gather(table, idx_flat):
    info = plsc.get_sparse_core_info()
    _NC, _NS = info.num_cores, info.num_subcores
    _NW = _NC * _NS
    _ROWS_PER_W = TOK // _NW
    mesh = plsc.VectorSubcoreMesh(core_axis_name="c", subcore_axis_name="s")

    @functools.partial(
        pl.kernel,
        out_type=jax.ShapeDtypeStruct((TOK, E_DIM), jnp.float32),
        mesh=mesh,
        scratch_types=[
            pltpu.VMEM((_CHUNK,), jnp.int32),
            pltpu.VMEM((_CHUNK,), jnp.int32),
            pltpu.VMEM((_ROWS_PER_W, E_DIM), jnp.float32),
            pltpu.SemaphoreType.DMA,
        ],
    )
    def gather(table_hbm, idx_hbm, out_hbm, idx_a, idx_b, rows_v, sem):
        wid = lax.axis_index("s") * _NC + lax.axis_index("c")
        base = wid * _ROWS_PER_W
        pltpu.sync_copy(idx_hbm.at[pl.ds(base, _CHUNK)], idx_a)
        pltpu.sync_copy(idx_hbm.at[pl.ds(base + _CHUNK, _CHUNK)], idx_b)
        cp1 = pltpu.async_copy(table_hbm.at[idx_a],
                               rows_v.at[pl.ds(0, _CHUNK)], sem)
        cp2 = pltpu.async_copy(table_hbm.at[idx_b],
                               rows_v.at[pl.ds(_CHUNK, _CHUNK)], sem)
        cp1.wait()
        cp2.wait()
        pltpu.sync_copy(rows_v, out_hbm.at[pl.ds(base, _ROWS_PER_W)])

    return gather(table, idx_flat)


def _final_body(z_ref, er_ref, w_ref, b_ref, out_ref, part_ref):
    zq = lax.dot_general(er_ref[0], w_ref[...], (((1,), (1,)), ((), ())),
                         precision=MM_PREC,
                         preferred_element_type=jnp.float32) + b_ref[...]
    zq_t = jnp.transpose(zq, (1, 0))
    z_b = z_ref[0]
    out_ref[0] = zq_t
    diff = zq_t - z_b
    part_ref[0, 0, 0] = jnp.sum(diff * diff)


def _finalize(z, emb_rows, proj_W, proj_b):
    return pl.pallas_call(
        _final_body,
        grid=(B,),
        in_specs=[
            pl.BlockSpec((1, C, H), lambda b: (b, 0, 0)),
            pl.BlockSpec((1, H, C), lambda b: (b, 0, 0)),
            pl.BlockSpec((E_DIM, E_DIM), lambda b: (0, 0)),
            pl.BlockSpec((1, E_DIM), lambda b: (0, 0)),
        ],
        out_specs=[
            pl.BlockSpec((1, C, H), lambda b: (b, 0, 0)),
            pl.BlockSpec((1, 1, 1), lambda b: (b, 0, 0),
                         memory_space=pltpu.SMEM),
        ],
        out_shape=[
            jax.ShapeDtypeStruct((B, C, H), jnp.float32),
            jax.ShapeDtypeStruct((B, 1, 1), jnp.float32),
        ],
    )(z, emb_rows.reshape(B, H, E_DIM), proj_W, proj_b.reshape(1, E_DIM))


def kernel(z, audio_domain, n_q, embedding, proj_W, proj_b):
    del n_q
    dom = audio_domain.astype(jnp.int32)
    offs = dom * (2048 // KB)
    width = jnp.where(dom == 2, 4096, 2048).astype(jnp.int32)

    idx = _search(z, embedding, proj_W, proj_b, offs, width).reshape(B, H)
    emb_rows = _sc_gather(embedding, idx.reshape(TOK))
    z_q, parts = _finalize(z, emb_rows, proj_W, proj_b)

    commit_loss = jnp.sum(parts) * ((1.0 + BETA) / (B * C * H))
    return (z_q, idx.reshape(1, B, H), commit_loss)


# speedup vs baseline: 1.0863x; 1.0863x over previous
import functools

import jax
import jax.numpy as jnp
from jax import lax
from jax.experimental import pallas as pl
from jax.experimental.pallas import tpu as pltpu
from jax.experimental.pallas import tpu_sc as plsc

N_E = 8192
E_DIM = 256
BETA = 0.25
B, C, H = 8, 256, 1024
TOK = B * H

KB = 1024
WIN = 4096
KSTEPS = WIN // KB
MM_PREC = lax.Precision.DEFAULT


def _argmin_body(offs_ref, width_ref, z_ref, emb_ref, w_ref, b_ref, idx_ref,
                 qc_ref, zn_ref, rmin_ref, ridx_ref):
    b = pl.program_id(0)
    k = pl.program_id(1)

    @pl.when((b == 0) & (k == 0))
    def _project():
        qc_ref[...] = lax.dot_general(
            emb_ref[...], w_ref[...], (((1,), (1,)), ((), ())),
            precision=MM_PREC,
            preferred_element_type=jnp.float32) + b_ref[...]

    @pl.when(k == 0)
    def _init():
        z_b = z_ref[0]
        zn_ref[...] = jnp.sum(z_b * z_b, axis=0, keepdims=True)
        rmin_ref[...] = jnp.full((1, H), jnp.inf, jnp.float32)
        ridx_ref[...] = jnp.zeros((1, H), jnp.float32)

    @pl.when(k * KB < width_ref[b])
    def _compute():
        start = (offs_ref[b] + k) * KB
        qc = qc_ref[pl.ds(start, KB), :]
        cbn = jnp.sum(qc * qc, axis=1, keepdims=True)
        mm = lax.dot_general(qc, z_ref[0], (((1,), (0,)), ((), ())),
                             precision=MM_PREC,
                             preferred_element_type=jnp.float32)
        d = (zn_ref[...] + cbn) - 2.0 * mm

        rows = (jnp.float32(k * KB)
                + lax.broadcasted_iota(jnp.int32, (KB, 1), 0).astype(jnp.float32))
        cmin = jnp.min(d, axis=0, keepdims=True)
        cidx = jnp.min(jnp.where(d == cmin, rows, jnp.float32(2**30)),
                       axis=0, keepdims=True)

        take = cmin < rmin_ref[...]
        rmin_ref[...] = jnp.where(take, cmin, rmin_ref[...])
        ridx_ref[...] = jnp.where(take, cidx, ridx_ref[...])

    @pl.when(k == KSTEPS - 1)
    def _emit():
        idx_ref[0] = ridx_ref[...].astype(jnp.int32) + offs_ref[b] * KB


def _search(z, embedding, proj_W, proj_b, offs, width):
    grid_spec = pltpu.PrefetchScalarGridSpec(
        num_scalar_prefetch=2,
        grid=(B, KSTEPS),
        in_specs=[
            pl.BlockSpec((1, C, H), lambda b, k, offs, width: (b, 0, 0)),
            pl.BlockSpec((N_E, E_DIM), lambda b, k, offs, width: (0, 0)),
            pl.BlockSpec((E_DIM, E_DIM), lambda b, k, offs, width: (0, 0)),
            pl.BlockSpec((1, E_DIM), lambda b, k, offs, width: (0, 0)),
        ],
        out_specs=pl.BlockSpec((1, 1, H), lambda b, k, offs, width: (b, 0, 0)),
        scratch_shapes=[
            pltpu.VMEM((N_E, E_DIM), jnp.float32),
            pltpu.VMEM((1, H), jnp.float32),
            pltpu.VMEM((1, H), jnp.float32),
            pltpu.VMEM((1, H), jnp.float32),
        ],
    )
    return pl.pallas_call(
        _argmin_body,
        grid_spec=grid_spec,
        out_shape=jax.ShapeDtypeStruct((B, 1, H), jnp.int32),
    )(offs, width, z, embedding, proj_W, proj_b.reshape(1, E_DIM))


_CHUNK = 128


def _sc_gather(table, idx_flat):
    info = plsc.get_sparse_core_info()
    _NC, _NS = info.num_cores, info.num_subcores
    _NW = _NC * _NS
    _ROWS_PER_W = TOK // _NW
    mesh = plsc.VectorSubcoreMesh(core_axis_name="c", subcore_axis_name="s")

    @functools.partial(
        pl.kernel,
        out_type=jax.ShapeDtypeStruct((TOK, E_DIM), jnp.float32),
        mesh=mesh,
        scratch_types=[
            pltpu.VMEM((_CHUNK,), jnp.int32),
            pltpu.VMEM((_CHUNK,), jnp.int32),
            pltpu.VMEM((_ROWS_PER_W, E_DIM), jnp.float32),
            pltpu.SemaphoreType.DMA,
        ],
    )
    def gather(table_hbm, idx_hbm, out_hbm, idx_a, idx_b, rows_v, sem):
        wid = lax.axis_index("s") * _NC + lax.axis_index("c")
        base = wid * _ROWS_PER_W
        pltpu.sync_copy(idx_hbm.at[pl.ds(base, _CHUNK)], idx_a)
        pltpu.sync_copy(idx_hbm.at[pl.ds(base + _CHUNK, _CHUNK)], idx_b)
        cp1 = pltpu.async_copy(table_hbm.at[idx_a],
                               rows_v.at[pl.ds(0, _CHUNK)], sem)
        cp2 = pltpu.async_copy(table_hbm.at[idx_b],
                               rows_v.at[pl.ds(_CHUNK, _CHUNK)], sem)
        cp1.wait()
        cp2.wait()
        pltpu.sync_copy(rows_v, out_hbm.at[pl.ds(base, _ROWS_PER_W)])

    return gather(table, idx_flat)


def _final_body(z_ref, er_ref, w_ref, b_ref, out_ref, part_ref):
    zq = lax.dot_general(er_ref[0], w_ref[...], (((1,), (1,)), ((), ())),
                         precision=MM_PREC,
                         preferred_element_type=jnp.float32) + b_ref[...]
    zq_t = jnp.transpose(zq, (1, 0))
    z_b = z_ref[0]
    out_ref[0] = zq_t
    diff = zq_t - z_b
    part_ref[0, 0, 0] = jnp.sum(diff * diff)


def _finalize(z, emb_rows, proj_W, proj_b):
    return pl.pallas_call(
        _final_body,
        grid=(B,),
        in_specs=[
            pl.BlockSpec((1, C, H), lambda b: (b, 0, 0)),
            pl.BlockSpec((1, H, C), lambda b: (b, 0, 0)),
            pl.BlockSpec((E_DIM, E_DIM), lambda b: (0, 0)),
            pl.BlockSpec((1, E_DIM), lambda b: (0, 0)),
        ],
        out_specs=[
            pl.BlockSpec((1, C, H), lambda b: (b, 0, 0)),
            pl.BlockSpec((1, 1, 1), lambda b: (b, 0, 0),
                         memory_space=pltpu.SMEM),
        ],
        out_shape=[
            jax.ShapeDtypeStruct((B, C, H), jnp.float32),
            jax.ShapeDtypeStruct((B, 1, 1), jnp.float32),
        ],
    )(z, emb_rows.reshape(B, H, E_DIM), proj_W, proj_b.reshape(1, E_DIM))


def kernel(z, audio_domain, n_q, embedding, proj_W, proj_b):
    del n_q
    dom = audio_domain.astype(jnp.int32)
    offs = dom * (2048 // KB)
    width = jnp.where(dom == 2, 4096, 2048).astype(jnp.int32)

    idx = _search(z, embedding, proj_W, proj_b, offs, width).reshape(B, H)
    emb_rows = _sc_gather(embedding, idx.reshape(TOK))
    z_q, parts = _finalize(z, emb_rows, proj_W, proj_b)

    commit_loss = jnp.sum(parts) * ((1.0 + BETA) / (B * C * H))
    return (z_q, idx.reshape(1, B, H), commit_loss)
